# baseline (speedup 1.0000x reference)
import functools

import jax
import jax.numpy as jnp
from jax import lax
from jax.experimental import pallas as pl
from jax.experimental.pallas import tpu as pltpu
from jax.experimental.pallas import tpu_sc as plsc

LANES = 16
CB = 384
V = 1000000
D = 64
VMAIN = 999936
NCHUNK = VMAIN // CB


def _mo(x, m):
    return pl.multiple_of(x, m)


@functools.lru_cache(maxsize=None)
def _make_transpose():
    info = plsc.get_sparse_core_info()
    NC, NS = info.num_cores, info.num_subcores
    NW = NC * NS
    per_w = NCHUNK // NW
    extra = NCHUNK - per_w * NW
    n_pairs = per_w // 2

    mesh = plsc.VectorSubcoreMesh(core_axis_name="c", subcore_axis_name="s")

    @functools.partial(
        pl.kernel,
        mesh=mesh,
        compiler_params=pltpu.CompilerParams(
            use_tc_tiling_on_sc=True, needs_layout_passes=False),
        out_type=jax.ShapeDtypeStruct((V // 2, 2 * D), jnp.float32),
        scratch_types=[
            pltpu.VMEM((D, CB), jnp.float32),
            pltpu.VMEM((D, CB), jnp.float32),
            pltpu.VMEM((CB // 2, 2 * D), jnp.float32),
            pltpu.VMEM((CB // 2, 2 * D), jnp.float32),
            pltpu.VMEM((D, D), jnp.float32),
            pltpu.SemaphoreType.DMA,
            pltpu.SemaphoreType.DMA,
            pltpu.SemaphoreType.DMA,
            pltpu.SemaphoreType.DMA,
            pltpu.SemaphoreType.DMA,
        ],
    )
    def transpose_kernel(tabT_hbm, tail_hbm, pt_hbm,
                         vin0, vin1, vout0, vout1, tl,
                         si0, si1, so0, so1, st):
        wid = lax.axis_index("s") * NC + lax.axis_index("c")
        vin = (vin0, vin1)
        vout = (vout0, vout1)
        si = (si0, si1)
        so = (so0, so1)
        lanes = lax.iota(jnp.int32, LANES)

        def chunk_off(ci):
            return (wid + NW * ci) * CB

        def read_copy(ci, b):
            return pltpu.make_async_copy(
                tabT_hbm.at[:, pl.ds(_mo(chunk_off(ci), 128), CB)],
                vin[b], si[b])

        def write_copy(ci, b):
            return pltpu.make_async_copy(
                vout[b],
                pt_hbm.at[pl.ds(_mo(chunk_off(ci) // 2, 8), CB // 2)],
                so[b])

        def transpose(b):
            def body(r, _):
                for kk in range(8):
                    vals = plsc.load_gather(
                        vin[b],
                        [16 * (kk % 4) + lanes,
                         jnp.broadcast_to(2 * r + kk // 4, (LANES,))])
                    vout[b][r, pl.ds(16 * kk, 16)] = vals
                return 0
            lax.fori_loop(0, CB // 2, body, 0)

        read_copy(0, 0).start()
        read_copy(1, 1).start()

        def pair(i2, _):
            for b in range(2):
                ci = 2 * i2 + b
                read_copy(ci, b).wait()

                @pl.when(i2 > 0)
                def _():
                    write_copy(ci, b).wait()
                transpose(b)

                @pl.when(ci + 2 < per_w)
                def _():
                    read_copy(ci + 2, b).start()
                write_copy(ci, b).start()
            return 0

        lax.fori_loop(0, n_pairs, pair, 0)

        ci = per_w - 1
        read_copy(ci, 0).wait()
        write_copy(ci, 0).wait()
        transpose(0)

        @pl.when(wid < extra)
        def _():
            read_copy(per_w, 0).start()
        wc = write_copy(ci, 0)
        wc.start()
        wc.wait()

        @pl.when(wid < extra)
        def _():
            read_copy(per_w, 0).wait()
            write_copy(per_w - 2, 1).wait()
            transpose(0)
            wc2 = write_copy(per_w, 0)
            wc2.start()
            wc2.wait()

        @pl.when(wid >= extra)
        def _():
            write_copy(per_w - 2, 1).wait()

        @pl.when(wid == 20)
        def _():
            pltpu.async_copy(tail_hbm, tl, st).wait()

            def tbody(r, _):
                for kk in range(8):
                    vals = plsc.load_gather(
                        tl,
                        [jnp.broadcast_to(2 * r + kk // 4, (LANES,)),
                         16 * (kk % 4) + lanes])
                    vout0[r, pl.ds(16 * kk, 16)] = vals
                return 0
            lax.fori_loop(0, 32, tbody, 0)
            pltpu.async_copy(
                vout0.at[pl.ds(0, 32)],
                pt_hbm.at[pl.ds(VMAIN // 2, 32)], st).wait()

    return transpose_kernel


@functools.lru_cache(maxsize=None)
def _make_gather(N, S):
    info = plsc.get_sparse_core_info()
    NC, NS = info.num_cores, info.num_subcores
    NW = NC * NS
    assert N % NW == 0
    n_per_w = N // NW
    b_per_w = n_per_w * S

    mesh = plsc.VectorSubcoreMesh(core_axis_name="c", subcore_axis_name="s")

    @functools.partial(
        pl.kernel,
        mesh=mesh,
        compiler_params=pltpu.CompilerParams(
            use_tc_tiling_on_sc=True, needs_layout_passes=False),
        out_type=jax.ShapeDtypeStruct((S, D, N), jnp.float32),
        scratch_types=[
            pltpu.VMEM((n_per_w, 128), jnp.int32),
            pltpu.VMEM((b_per_w,), jnp.int32),
            pltpu.VMEM((b_per_w,), jnp.int32),
            pltpu.VMEM((n_per_w, 128), jnp.float32),
            pltpu.VMEM((n_per_w, 128), jnp.float32),
            pltpu.VMEM((D, n_per_w), jnp.float32),
            pltpu.VMEM((D, n_per_w), jnp.float32),
            pltpu.SemaphoreType.DMA,
            pltpu.SemaphoreType.DMA,
            pltpu.SemaphoreType.DMA,
            pltpu.SemaphoreType.DMA,
            pltpu.SemaphoreType.DMA,
        ],
    )
    def gather_kernel(idx_hbm, pt_hbm, out_hbm,
                      idx2d, idxs, hbuf, rows0, rows1, tr0, tr1,
                      si, sg0, sg1, so0, so1):
        wid = lax.axis_index("s") * NC + lax.axis_index("c")
        base_n = wid * n_per_w
        rows = (rows0, rows1)
        trans = (tr0, tr1)
        sg = (sg0, sg1)
        so = (so0, so1)
        lanes = lax.iota(jnp.int32, LANES)

        pltpu.async_copy(
            idx_hbm.at[pl.ds(_mo(base_n, 8), n_per_w)], idx2d, si).wait()

        def rep(m, _):
            s = m // 8
            npr = 16 * (m % 8) + lanes
            vals = plsc.load_gather(
                idx2d, [npr, jnp.broadcast_to(s, (LANES,))])
            off16 = pl.ds(_mo(m * 16, 16), 16)
            idxs[off16] = vals >> 1
            hbuf[off16] = (vals & 1) * D
            return 0

        lax.fori_loop(0, b_per_w // LANES, rep, 0)

        def gather_copy(s, b):
            return pltpu.make_async_copy(
                pt_hbm.at[idxs.at[pl.ds(_mo(s * n_per_w, 8), n_per_w)]],
                rows[b], sg[b])

        def write_copy(s, b):
            return pltpu.make_async_copy(
                trans[b],
                out_hbm.at[s, :, pl.ds(_mo(base_n, 128), n_per_w)],
                so[b])

        def transpose(s, b):
            def tbody(d, _):
                for kk in range(8):
                    hv = hbuf[pl.ds(_mo(s * n_per_w + 16 * kk, 16), 16)]
                    vals = plsc.load_gather(
                        rows[b],
                        [16 * kk + lanes, hv + jnp.broadcast_to(d, (LANES,))])
                    trans[b][d, pl.ds(16 * kk, 16)] = vals
                return 0
            lax.fori_loop(0, D, tbody, 0)

        gather_copy(0, 0).start()
        gather_copy(1, 1).start()

        def pair(i2, _):
            for b in range(2):
                s = 2 * i2 + b
                gather_copy(s, b).wait()

                @pl.when(i2 > 0)
                def _():
                    write_copy(s, b).wait()
                transpose(s, b)

                @pl.when(s + 2 < S)
                def _():
                    gather_copy(s + 2, b).start()
                write_copy(s, b).start()
            return 0

        lax.fori_loop(0, S // 2, pair, 0)
        write_copy(S - 2, 0).wait()
        write_copy(S - 1, 1).wait()

    return gather_kernel


def kernel(idx, embeddings):
    n, s = idx.shape
    idx_p = jnp.pad(idx.astype(jnp.int32), ((0, 0), (0, 128 - s)))
    tab_t = embeddings.T
    tail = embeddings[VMAIN:, :]
    pt = _make_transpose()(tab_t, tail)
    out = _make_gather(n, s)(idx_p, pt)
    return out.transpose(2, 0, 1)


# speedup vs baseline: 1.4842x; 1.4842x over previous
import functools

import jax
import jax.numpy as jnp
from jax import lax
from jax.experimental import pallas as pl
from jax.experimental.pallas import tpu as pltpu
from jax.experimental.pallas import tpu_sc as plsc

LANES = 16
CB = 384
V = 1000000
D = 64
VMAIN = 999936
NCHUNK = VMAIN // CB


def _mo(x, m):
    return pl.multiple_of(x, m)


@functools.lru_cache(maxsize=None)
def _make_transpose():
    info = plsc.get_sparse_core_info()
    NC, NS = info.num_cores, info.num_subcores
    NW = NC * NS
    per_w = NCHUNK // NW
    extra = NCHUNK - per_w * NW
    n_pairs = per_w // 2

    mesh = plsc.VectorSubcoreMesh(core_axis_name="c", subcore_axis_name="s")

    @functools.partial(
        pl.kernel,
        mesh=mesh,
        compiler_params=pltpu.CompilerParams(
            use_tc_tiling_on_sc=True, needs_layout_passes=False),
        out_type=jax.ShapeDtypeStruct((V // 2, 2 * D), jnp.float32),
        scratch_types=[
            pltpu.VMEM((D, CB), jnp.float32),
            pltpu.VMEM((D, CB), jnp.float32),
            pltpu.VMEM((CB // 2, 2 * D), jnp.float32),
            pltpu.VMEM((CB // 2, 2 * D), jnp.float32),
            pltpu.VMEM((D, D), jnp.float32),
            pltpu.SemaphoreType.DMA,
            pltpu.SemaphoreType.DMA,
            pltpu.SemaphoreType.DMA,
            pltpu.SemaphoreType.DMA,
            pltpu.SemaphoreType.DMA,
        ],
    )
    def transpose_kernel(tabT_hbm, tail_hbm, pt_hbm,
                         vin0, vin1, vout0, vout1, tl,
                         si0, si1, so0, so1, st):
        wid = lax.axis_index("s") * NC + lax.axis_index("c")
        vin = (vin0, vin1)
        vout = (vout0, vout1)
        si = (si0, si1)
        so = (so0, so1)
        lanes = lax.iota(jnp.int32, LANES)

        def chunk_off(ci):
            return (wid + NW * ci) * CB

        def read_copy(ci, b):
            return pltpu.make_async_copy(
                tabT_hbm.at[:, pl.ds(_mo(chunk_off(ci), 128), CB)],
                vin[b], si[b])

        def write_copy(ci, b):
            return pltpu.make_async_copy(
                vout[b],
                pt_hbm.at[pl.ds(_mo(chunk_off(ci) // 2, 8), CB // 2)],
                so[b])

        def transpose(b):
            def body(r2, _):
                r = 2 * r2
                vals = [
                    plsc.load_gather(
                        vin[b],
                        [16 * (kk % 4) + lanes,
                         jnp.broadcast_to(2 * r + (kk % 8) // 4 + 2 * (kk // 8),
                                          (LANES,))])
                    for kk in range(16)
                ]
                for kk in range(16):
                    vout[b][r + kk // 8, pl.ds(16 * (kk % 8), 16)] = vals[kk]
                return 0
            lax.fori_loop(0, CB // 4, body, 0)

        read_copy(0, 0).start()
        read_copy(1, 1).start()

        def pair(i2, _):
            for b in range(2):
                ci = 2 * i2 + b
                read_copy(ci, b).wait()

                @pl.when(i2 > 0)
                def _():
                    write_copy(ci, b).wait()
                transpose(b)

                @pl.when(ci + 2 < per_w)
                def _():
                    read_copy(ci + 2, b).start()
                write_copy(ci, b).start()
            return 0

        lax.fori_loop(0, n_pairs, pair, 0)

        ci = per_w - 1
        read_copy(ci, 0).wait()
        write_copy(ci, 0).wait()
        transpose(0)

        @pl.when(wid < extra)
        def _():
            read_copy(per_w, 0).start()
        wc = write_copy(ci, 0)
        wc.start()
        wc.wait()

        @pl.when(wid < extra)
        def _():
            read_copy(per_w, 0).wait()
            write_copy(per_w - 2, 1).wait()
            transpose(0)
            wc2 = write_copy(per_w, 0)
            wc2.start()
            wc2.wait()

        @pl.when(wid >= extra)
        def _():
            write_copy(per_w - 2, 1).wait()

        @pl.when(wid == 20)
        def _():
            pltpu.async_copy(tail_hbm, tl, st).wait()

            def tbody(r, _):
                vals = [
                    plsc.load_gather(
                        tl,
                        [jnp.broadcast_to(2 * r + kk // 4, (LANES,)),
                         16 * (kk % 4) + lanes])
                    for kk in range(8)
                ]
                for kk in range(8):
                    vout0[r, pl.ds(16 * kk, 16)] = vals[kk]
                return 0
            lax.fori_loop(0, 32, tbody, 0)
            pltpu.async_copy(
                vout0.at[pl.ds(0, 32)],
                pt_hbm.at[pl.ds(VMAIN // 2, 32)], st).wait()

    return transpose_kernel


@functools.lru_cache(maxsize=None)
def _make_gather(N, S):
    info = plsc.get_sparse_core_info()
    NC, NS = info.num_cores, info.num_subcores
    NW = NC * NS
    assert N % NW == 0
    n_per_w = N // NW
    b_per_w = n_per_w * S

    mesh = plsc.VectorSubcoreMesh(core_axis_name="c", subcore_axis_name="s")

    @functools.partial(
        pl.kernel,
        mesh=mesh,
        compiler_params=pltpu.CompilerParams(
            use_tc_tiling_on_sc=True, needs_layout_passes=False),
        out_type=jax.ShapeDtypeStruct((S, D, N), jnp.float32),
        scratch_types=[
            pltpu.VMEM((n_per_w, 128), jnp.int32),
            pltpu.VMEM((b_per_w,), jnp.int32),
            pltpu.VMEM((b_per_w,), jnp.int32),
            pltpu.VMEM((n_per_w, 128), jnp.float32),
            pltpu.VMEM((n_per_w, 128), jnp.float32),
            pltpu.VMEM((D, n_per_w), jnp.float32),
            pltpu.VMEM((D, n_per_w), jnp.float32),
            pltpu.SemaphoreType.DMA,
            pltpu.SemaphoreType.DMA,
            pltpu.SemaphoreType.DMA,
            pltpu.SemaphoreType.DMA,
            pltpu.SemaphoreType.DMA,
        ],
    )
    def gather_kernel(idx_hbm, pt_hbm, out_hbm,
                      idx2d, idxs, hbuf, rows0, rows1, tr0, tr1,
                      si, sg0, sg1, so0, so1):
        wid = lax.axis_index("s") * NC + lax.axis_index("c")
        base_n = wid * n_per_w
        rows = (rows0, rows1)
        trans = (tr0, tr1)
        sg = (sg0, sg1)
        so = (so0, so1)
        lanes = lax.iota(jnp.int32, LANES)

        pltpu.async_copy(
            idx_hbm.at[pl.ds(_mo(base_n, 8), n_per_w)], idx2d, si).wait()

        def rep(m, _):
            s = m // 8
            npr = 16 * (m % 8) + lanes
            vals = plsc.load_gather(
                idx2d, [npr, jnp.broadcast_to(s, (LANES,))])
            off16 = pl.ds(_mo(m * 16, 16), 16)
            idxs[off16] = vals >> 1
            hbuf[off16] = (vals & 1) * D
            return 0

        lax.fori_loop(0, b_per_w // LANES, rep, 0)

        def gather_copy(s, b):
            return pltpu.make_async_copy(
                pt_hbm.at[idxs.at[pl.ds(_mo(s * n_per_w, 8), n_per_w)]],
                rows[b], sg[b])

        def write_copy(s, b):
            return pltpu.make_async_copy(
                trans[b],
                out_hbm.at[s, :, pl.ds(_mo(base_n, 128), n_per_w)],
                so[b])

        def transpose(s, b):
            hvs = [hbuf[pl.ds(_mo(s * n_per_w + 16 * kk, 16), 16)]
                   for kk in range(8)]

            def tbody(d, _):
                vals = [
                    plsc.load_gather(
                        rows[b],
                        [16 * kk + lanes,
                         hvs[kk] + jnp.broadcast_to(d, (LANES,))])
                    for kk in range(8)
                ]
                for kk in range(8):
                    trans[b][d, pl.ds(16 * kk, 16)] = vals[kk]
                return 0
            lax.fori_loop(0, D, tbody, 0)

        gather_copy(0, 0).start()
        gather_copy(1, 1).start()

        def pair(i2, _):
            for b in range(2):
                s = 2 * i2 + b
                gather_copy(s, b).wait()

                @pl.when(i2 > 0)
                def _():
                    write_copy(s, b).wait()
                transpose(s, b)

                @pl.when(s + 2 < S)
                def _():
                    gather_copy(s + 2, b).start()
                write_copy(s, b).start()
            return 0

        lax.fori_loop(0, S // 2, pair, 0)
        write_copy(S - 2, 0).wait()
        write_copy(S - 1, 1).wait()

    return gather_kernel


def kernel(idx, embeddings):
    n, s = idx.shape
    idx_p = jnp.pad(idx.astype(jnp.int32), ((0, 0), (0, 128 - s)))
    tab_t = embeddings.T
    tail = embeddings[VMAIN:, :]
    pt = _make_transpose()(tab_t, tail)
    out = _make_gather(n, s)(idx_p, pt)
    return out.transpose(2, 0, 1)


# speedup vs baseline: 2.5748x; 1.7348x over previous
import functools

import jax
import jax.numpy as jnp
from jax import lax
from jax.experimental import pallas as pl
from jax.experimental.pallas import tpu as pltpu
from jax.experimental.pallas import tpu_sc as plsc

EMBED_DIM = 64


@functools.lru_cache(maxsize=None)
def _make_gather(B, D):
    info = plsc.get_sparse_core_info()
    NC, NS = info.num_cores, info.num_subcores
    NW = NC * NS
    assert B % NW == 0
    b_per_w = B // NW
    CH = 800
    assert b_per_w % CH == 0
    n_ch = b_per_w // CH

    mesh = plsc.VectorSubcoreMesh(core_axis_name="c", subcore_axis_name="s")

    @functools.partial(
        pl.kernel,
        mesh=mesh,
        compiler_params=pltpu.CompilerParams(use_tc_tiling_on_sc=False),
        out_type=jax.ShapeDtypeStruct((B, D), jnp.float32),
        scratch_types=[
            pltpu.VMEM((CH,), jnp.int32),
            pltpu.VMEM((CH,), jnp.int32),
            pltpu.VMEM((CH, D), jnp.float32),
            pltpu.VMEM((CH, D), jnp.float32),
            pltpu.SemaphoreType.DMA,
            pltpu.SemaphoreType.DMA,
            pltpu.SemaphoreType.DMA,
            pltpu.SemaphoreType.DMA,
            pltpu.SemaphoreType.DMA,
            pltpu.SemaphoreType.DMA,
        ],
    )
    def gather_kernel(idx_hbm, table_hbm, out_hbm,
                      idx_v0, idx_v1, rows_v0, rows_v1,
                      si0, si1, sg0, sg1, so0, so1):
        wid = lax.axis_index("s") * NC + lax.axis_index("c")
        base = wid * b_per_w
        idx_bufs = (idx_v0, idx_v1)
        rows_bufs = (rows_v0, rows_v1)
        si = (si0, si1)
        sg = (sg0, sg1)
        so = (so0, so1)
        copies_i = [None, None]
        copies_o = [None, None]
        copies_i[0] = pltpu.async_copy(
            idx_hbm.at[pl.ds(base, CH)], idx_bufs[0], si[0])
        for j in range(n_ch):
            b = j % 2
            if j + 1 < n_ch:
                nb = (j + 1) % 2
                copies_i[nb] = pltpu.async_copy(
                    idx_hbm.at[pl.ds(base + (j + 1) * CH, CH)],
                    idx_bufs[nb], si[nb])
            copies_i[b].wait()
            if copies_o[b] is not None:
                copies_o[b].wait()
            gather = pltpu.async_copy(
                table_hbm.at[idx_bufs[b]], rows_bufs[b], sg[b])
            gather.wait()
            copies_o[b] = pltpu.async_copy(
                rows_bufs[b], out_hbm.at[pl.ds(base + j * CH, CH)], so[b])
        copies_o[(n_ch - 2) % 2].wait()
        copies_o[(n_ch - 1) % 2].wait()

    return gather_kernel


def kernel(idx, embeddings):
    n, s = idx.shape
    flat = idx.reshape(n * s).astype(jnp.int32)
    out = _make_gather(n * s, EMBED_DIM)(flat, embeddings)
    return out.reshape(n, s, EMBED_DIM)
